# Initial kernel scaffold; baseline (speedup 1.0000x reference)
import jax
import jax.numpy as jnp
from jax.experimental import pallas as pl


def kernel(feature, edge_index, W_f1, b_f1, W_e1, b_e1, W_a1, b_a1, watt1, batt1, W_e2, b_e2, W_a2, b_a2, watt2, batt2):
    raise NotImplementedError("write your pallas kernel here")


# speedup vs baseline: 7.4828x; 7.4828x over previous
import functools

import jax
import jax.numpy as jnp
from jax import lax
from jax.experimental import pallas as pl
from jax.experimental.pallas import tpu as pltpu
from jax.experimental.pallas import tpu_sc as plsc

N = 10000
E = 320000
D = 128
C1 = 100
C2 = 10

WA = 112
WH = 144
NC, NS = 2, 16
NW = NC * NS
EPW = E // NW
K = 80
NCHUNK = EPW // K
NPAD = 10240
RPT = NPAD // NS

_SC_PARAMS = pltpu.CompilerParams(use_tc_tiling_on_sc=False,
                                  needs_layout_passes=False)

_BLK = 1000
_NBLK = N // _BLK


def _seg_sum_call(table, gidx, sidx, width):
    mesh = plsc.VectorSubcoreMesh(core_axis_name="c", subcore_axis_name="s")

    @functools.partial(
        pl.kernel,
        mesh=mesh,
        out_type=jax.ShapeDtypeStruct((NC, NPAD, width), jnp.float32),
        compiler_params=_SC_PARAMS,
        scratch_types=[
            pltpu.VMEM((K,), jnp.int32),
            pltpu.VMEM((K,), jnp.int32),
            pltpu.VMEM((K, width), jnp.float32),
            pltpu.VMEM_SHARED((NPAD, width), jnp.float32),
            pltpu.SemaphoreType.DMA,
        ],
    )
    def seg(table_hbm, gidx_hbm, sidx_hbm, zeros_hbm, out_hbm,
            gix_v, six_v, rows_v, acc_sh, sem):
        cid = lax.axis_index("c")
        sid = lax.axis_index("s")
        base = (cid * NS + sid) * EPW
        pltpu.sync_copy(zeros_hbm, acc_sh.at[pl.ds(sid * RPT, RPT)])
        plsc.subcore_barrier()

        def body(ci, carry):
            off = pl.multiple_of(base + ci * K, 16)
            pltpu.sync_copy(gidx_hbm.at[pl.ds(off, K)], gix_v)
            pltpu.sync_copy(sidx_hbm.at[pl.ds(off, K)], six_v)
            pltpu.async_copy(table_hbm.at[gix_v], rows_v, sem).wait()
            pltpu.sync_copy(rows_v, acc_sh.at[six_v], add=True)
            return carry

        lax.fori_loop(0, NCHUNK, body, 0)
        plsc.subcore_barrier()
        pltpu.sync_copy(acc_sh.at[pl.ds(sid * RPT, RPT)],
                        out_hbm.at[cid, pl.ds(sid * RPT, RPT)])

    zeros = jnp.zeros((RPT, width), jnp.float32)
    return seg(table, gidx, sidx, zeros)


def _att_seg_call(ha_aug, ad_pad, src, dst):
    mesh = plsc.VectorSubcoreMesh(core_axis_name="c", subcore_axis_name="s")

    @functools.partial(
        pl.kernel,
        mesh=mesh,
        out_type=jax.ShapeDtypeStruct((NC, NPAD, WA), jnp.float32),
        compiler_params=_SC_PARAMS,
        scratch_types=[
            pltpu.VMEM((K,), jnp.int32),
            pltpu.VMEM((K,), jnp.int32),
            pltpu.VMEM((K, WA), jnp.float32),
            pltpu.VMEM((K, 16), jnp.float32),
            pltpu.VMEM_SHARED((NPAD, WA), jnp.float32),
            pltpu.SemaphoreType.DMA,
            pltpu.SemaphoreType.DMA,
        ],
    )
    def att(ha_hbm, ad_hbm, src_hbm, dst_hbm, zeros_hbm, out_hbm,
            gix_v, six_v, rows_v, adr_v, acc_sh, sem1, sem2):
        cid = lax.axis_index("c")
        sid = lax.axis_index("s")
        base = (cid * NS + sid) * EPW
        pltpu.sync_copy(zeros_hbm, acc_sh.at[pl.ds(sid * RPT, RPT)])
        plsc.subcore_barrier()

        def body(ci, carry):
            off = pl.multiple_of(base + ci * K, 16)
            pltpu.sync_copy(src_hbm.at[pl.ds(off, K)], gix_v)
            pltpu.sync_copy(dst_hbm.at[pl.ds(off, K)], six_v)
            ga = pltpu.async_copy(ha_hbm.at[gix_v], rows_v, sem1)
            gb = pltpu.async_copy(ad_hbm.at[six_v], adr_v, sem2)
            ga.wait()
            gb.wait()
            for g in range(K // 16):
                rid = g * 16 + jnp.arange(16, dtype=jnp.int32)
                a_s = plsc.load_gather(
                    rows_v, [rid, jnp.full((16,), C1, jnp.int32)])
                a_d = plsc.load_gather(
                    adr_v, [rid, jnp.zeros((16,), jnp.int32)])
                e = a_s + a_d
                e = jnp.where(e > 0, e, 0.01 * e)
                ex16 = jnp.exp(e)
                for l in range(16):
                    r = g * 16 + l
                    svec = jnp.full((16,), ex16[l])
                    for j in range(WA // 16):
                        rows_v[r, pl.ds(j * 16, 16)] = (
                            rows_v[r, pl.ds(j * 16, 16)] * svec)
            pltpu.sync_copy(rows_v, acc_sh.at[six_v], add=True)
            return carry

        lax.fori_loop(0, NCHUNK, body, 0)
        plsc.subcore_barrier()
        pltpu.sync_copy(acc_sh.at[pl.ds(sid * RPT, RPT)],
                        out_hbm.at[cid, pl.ds(sid * RPT, RPT)])

    zeros = jnp.zeros((RPT, WA), jnp.float32)
    return att(ha_aug, ad_pad, src, dst, zeros)


def _tc1_body(feat, w, b, out):
    h = jnp.dot(feat[...], w[...], preferred_element_type=jnp.float32) + b[...]
    ones = jnp.ones((_BLK, 1), jnp.float32)
    zpad = jnp.zeros((_BLK, WH - D - 1), jnp.float32)
    out[...] = jnp.concatenate([h, ones, zpad], axis=1)


def _tc2_body(acc, h1a, we, be, x_out, h2_out, deg_out):
    agg = acc[0, :, :D] + acc[1, :, :D]
    deg = acc[0, :, D:D + 1] + acc[1, :, D:D + 1]
    h1 = h1a[:, :D]
    xm = jnp.where(deg > 0, agg / jnp.maximum(deg, 1.0), h1)
    nrm = jnp.sqrt(jnp.sum(xm * xm, axis=1, keepdims=True))
    x = xm / jnp.maximum(nrm, 1e-12)
    x_out[...] = x
    h2_out[...] = jnp.dot(x, we[...], preferred_element_type=jnp.float32) + be[...]
    deg_out[...] = deg


def _tc3_body(acc, deg_in, h2, wa, ba, wsrc, wdst, batt, z_out, haug_out, ad_out):
    deg = deg_in[...]
    agg = acc[0] + acc[1]
    z = jnp.where(deg > 0, agg / jnp.maximum(deg, 1.0), h2[...])
    z_out[...] = z
    ha = jnp.dot(z, wa[...], preferred_element_type=jnp.float32) + ba[...]
    a_s = jnp.dot(ha, wsrc[...], preferred_element_type=jnp.float32)
    a_d = jnp.dot(ha, wdst[...], preferred_element_type=jnp.float32) + batt[...]
    ones = jnp.ones((_BLK, 1), jnp.float32)
    zpad = jnp.zeros((_BLK, WA - C1 - 2), jnp.float32)
    haug_out[...] = jnp.concatenate([ha, a_s, ones, zpad], axis=1)
    ad_out[...] = jnp.concatenate([a_d, jnp.zeros((_BLK, 15), jnp.float32)], axis=1)


def _tc4_body(acc, deg_in, haug, spad_out):
    deg = deg_in[...]
    num = acc[0, :, :C1] + acc[1, :, :C1]
    den = acc[0, :, C1 + 1:C1 + 2] + acc[1, :, C1 + 1:C1 + 2]
    ha = haug[:, :C1]
    att = jnp.where(deg > 0, num / jnp.where(den > 0, den, 1.0), ha)
    m = jnp.max(att, axis=1, keepdims=True)
    ex = jnp.exp(att - m)
    s = ex / jnp.sum(ex, axis=1, keepdims=True)
    spad_out[...] = jnp.concatenate(
        [s, jnp.zeros((_BLK, WA - C1), jnp.float32)], axis=1)


def _tc4b_body(spad, z, tacc, x2_out, adj_out):
    i = pl.program_id(0)

    @pl.when(i == 0)
    def _():
        x2_out[...] = jnp.zeros_like(x2_out)
        adj_out[...] = jnp.zeros_like(adj_out)

    s_blk = spad[...]
    t_blk = tacc[0] + tacc[1]
    dn = (((0,), (0,)), ((), ()))
    x2_out[...] += lax.dot_general(s_blk, z[...], dn,
                                   preferred_element_type=jnp.float32)
    adj_out[...] += lax.dot_general(s_blk, t_blk, dn,
                                    preferred_element_type=jnp.float32)


def _tc5_body(x2, adj_in, we2, be2, wa2, ba2, w2s, w2d, batt2,
              s2_out, x3_out, emb_out):
    adj = adj_in[...]
    ii = lax.broadcasted_iota(jnp.int32, (C1, C1), 0)
    jj = lax.broadcasted_iota(jnp.int32, (C1, C1), 1)
    eye = (ii == jj)
    adj = jnp.where(eye, 0.0, adj)
    m1 = jnp.maximum((adj != 0).astype(jnp.float32),
                     eye.astype(jnp.float32))
    h = jnp.dot(x2[...], we2[...], preferred_element_type=jnp.float32) + be2[...]
    indeg = jnp.sum(m1, axis=0, keepdims=True)
    dn = (((0,), (0,)), ((), ()))
    aggd = lax.dot_general(m1, h, dn, preferred_element_type=jnp.float32)
    indeg_c = indeg.reshape(C1, 1)
    z2 = jnp.where(indeg_c > 0, aggd / jnp.maximum(indeg_c, 1.0), h)
    ha2 = jnp.dot(z2, wa2[...], preferred_element_type=jnp.float32) + ba2[...]
    a_s = jnp.dot(ha2, w2s[...], preferred_element_type=jnp.float32)
    a_d = jnp.dot(ha2, w2d[...], preferred_element_type=jnp.float32)
    a = a_s + a_d.reshape(1, C1) + batt2[...]
    e = jnp.where(a > 0, a, 0.01 * a)
    e = jnp.where(m1 > 0, e, -1e9)
    mcol = jnp.max(e, axis=0, keepdims=True)
    exv = jnp.exp(e - mcol)
    alpha = exv / jnp.sum(exv, axis=0, keepdims=True)
    outa = lax.dot_general(alpha, ha2, dn, preferred_element_type=jnp.float32)
    att2 = jnp.where(indeg_c > 0, outa, ha2)
    m2 = jnp.max(att2, axis=1, keepdims=True)
    ex2 = jnp.exp(att2 - m2)
    s2 = ex2 / jnp.sum(ex2, axis=1, keepdims=True)
    s2_out[...] = s2
    x3 = lax.dot_general(s2, z2, dn, preferred_element_type=jnp.float32)
    x3_out[...] = x3
    emb_out[...] = jnp.mean(x3).reshape(1, 1)


def _row_spec(width):
    return pl.BlockSpec((_BLK, width), lambda i: (i, 0))


def _full_spec(shape):
    nd = len(shape)
    return pl.BlockSpec(shape, lambda i, _n=nd: (0,) * _n)


def _acc_spec(width):
    return pl.BlockSpec((NC, _BLK, width), lambda i: (0, i, 0))


def kernel(feature, edge_index, W_f1, b_f1, W_e1, b_e1, W_a1, b_a1, watt1,
           batt1, W_e2, b_e2, W_a2, b_a2, watt2, batt2):
    src = edge_index[0]
    dst = edge_index[1]

    h1a = pl.pallas_call(
        _tc1_body,
        grid=(_NBLK,),
        in_specs=[_row_spec(D), _full_spec((D, D)), _full_spec((1, D))],
        out_specs=_row_spec(WH),
        out_shape=jax.ShapeDtypeStruct((N, WH), jnp.float32),
    )(feature, W_f1, b_f1.reshape(1, D))

    accA = _seg_sum_call(h1a, src, dst, WH)

    x, h2, deg = pl.pallas_call(
        _tc2_body,
        grid=(_NBLK,),
        in_specs=[_acc_spec(WH), _row_spec(WH), _full_spec((D, D)),
                  _full_spec((1, D))],
        out_specs=[_row_spec(D), _row_spec(D), _row_spec(1)],
        out_shape=[jax.ShapeDtypeStruct((N, D), jnp.float32),
                   jax.ShapeDtypeStruct((N, D), jnp.float32),
                   jax.ShapeDtypeStruct((N, 1), jnp.float32)],
    )(accA, h1a, W_e1, b_e1.reshape(1, D))

    accB = _seg_sum_call(h2, src, dst, D)

    z, ha_aug, ad_pad = pl.pallas_call(
        _tc3_body,
        grid=(_NBLK,),
        in_specs=[_acc_spec(D), _row_spec(1), _row_spec(D),
                  _full_spec((D, C1)), _full_spec((1, C1)),
                  _full_spec((C1, 1)), _full_spec((C1, 1)),
                  _full_spec((1, 1))],
        out_specs=[_row_spec(D), _row_spec(WA), _row_spec(16)],
        out_shape=[jax.ShapeDtypeStruct((N, D), jnp.float32),
                   jax.ShapeDtypeStruct((N, WA), jnp.float32),
                   jax.ShapeDtypeStruct((N, 16), jnp.float32)],
    )(accB, deg, h2, W_a1, b_a1.reshape(1, C1),
      watt1[:C1].reshape(C1, 1), watt1[C1:].reshape(C1, 1),
      batt1.reshape(1, 1))

    accC = _att_seg_call(ha_aug, ad_pad, src, dst)

    s_pad = pl.pallas_call(
        _tc4_body,
        grid=(_NBLK,),
        in_specs=[_acc_spec(WA), _row_spec(1), _row_spec(WA)],
        out_specs=_row_spec(WA),
        out_shape=jax.ShapeDtypeStruct((N, WA), jnp.float32),
    )(accC, deg, ha_aug)

    accT = _seg_sum_call(s_pad, dst, src, WA)

    x2f, adjf = pl.pallas_call(
        _tc4b_body,
        grid=(_NBLK,),
        in_specs=[_row_spec(WA), _row_spec(D), _acc_spec(WA)],
        out_specs=[_full_spec((WA, D)), _full_spec((WA, WA))],
        out_shape=[jax.ShapeDtypeStruct((WA, D), jnp.float32),
                   jax.ShapeDtypeStruct((WA, WA), jnp.float32)],
    )(s_pad, z, accT)

    s2, x3, emb = pl.pallas_call(
        _tc5_body,
        grid=(1,),
        in_specs=[_full_spec((C1, D)), _full_spec((C1, C1)),
                  _full_spec((D, D)), _full_spec((1, D)),
                  _full_spec((D, C2)), _full_spec((1, C2)),
                  _full_spec((C2, 1)), _full_spec((C2, 1)),
                  _full_spec((1, 1))],
        out_specs=[_full_spec((C1, C2)), _full_spec((C2, D)),
                   _full_spec((1, 1))],
        out_shape=[jax.ShapeDtypeStruct((C1, C2), jnp.float32),
                   jax.ShapeDtypeStruct((C2, D), jnp.float32),
                   jax.ShapeDtypeStruct((1, 1), jnp.float32)],
    )(x2f[:C1], adjf[:C1, :C1], W_e2, b_e2.reshape(1, D),
      W_a2, b_a2.reshape(1, C2), watt2[:C2].reshape(C2, 1),
      watt2[C2:].reshape(C2, 1), batt2.reshape(1, 1))

    s = s_pad[:, :C1]
    x2 = x2f[:C1]
    assign1 = jnp.ones((C2, 1), jnp.float32)
    return (s, s2, assign1, x, x2, x3, emb[0, 0])


# speedup vs baseline: 13.9131x; 1.8594x over previous
import functools

import jax
import jax.numpy as jnp
from jax import lax
from jax.experimental import pallas as pl
from jax.experimental.pallas import tpu as pltpu
from jax.experimental.pallas import tpu_sc as plsc

N = 10000
E = 320000
D = 128
C1 = 100
C2 = 10

WA = 112
WH = 144
NC, NS = 2, 16
NW = NC * NS
EPW = E // NW
KS = 100
KA = 80
NPAD = 10240
RPT = NPAD // NS

_SC_PARAMS = pltpu.CompilerParams(use_tc_tiling_on_sc=False,
                                  needs_layout_passes=False)

_BLK = 1000
_NBLK = N // _BLK


def _edge_pass(table, gidx2, sidx2, width, k, ad=None):
    weighted = ad is not None
    ncs = EPW // k
    npair = ncs // 2
    tail = ncs % 2
    mesh = plsc.VectorSubcoreMesh(core_axis_name="c", subcore_axis_name="s")

    scratch = [
        pltpu.VMEM((4, k), jnp.int32),
        pltpu.VMEM((4, k), jnp.int32),
        pltpu.VMEM((k, width), jnp.float32),
        pltpu.VMEM((k, width), jnp.float32),
        pltpu.VMEM_SHARED((NPAD, width), jnp.float32),
        pltpu.SemaphoreType.DMA,
        pltpu.SemaphoreType.DMA,
        pltpu.SemaphoreType.DMA,
        pltpu.SemaphoreType.DMA,
        pltpu.SemaphoreType.DMA,
        pltpu.SemaphoreType.DMA,
    ]
    if weighted:
        scratch += [
            pltpu.VMEM((k, 16), jnp.float32),
            pltpu.VMEM((k, 16), jnp.float32),
            pltpu.SemaphoreType.DMA,
            pltpu.SemaphoreType.DMA,
        ]

    def body(*refs):
        if weighted:
            (table_hbm, ad_hbm, gidx_hbm, sidx_hbm, zeros_hbm, out_hbm,
             gring, sring, rows0, rows1, acc_sh,
             gsem0, gsem1, ssem0, ssem1, isem0, isem1,
             adr0, adr1, asem0, asem1) = refs
        else:
            (table_hbm, gidx_hbm, sidx_hbm, zeros_hbm, out_hbm,
             gring, sring, rows0, rows1, acc_sh,
             gsem0, gsem1, ssem0, ssem1, isem0, isem1) = refs
            adr0 = adr1 = asem0 = asem1 = None
        cid = lax.axis_index("c")
        sid = lax.axis_index("s")
        w = cid * NS + sid
        crow0 = w * ncs

        pltpu.sync_copy(zeros_hbm, acc_sh.at[pl.ds(sid * RPT, RPT)])
        for c in range(4):
            pltpu.sync_copy(gidx_hbm.at[crow0 + c], gring.at[c])
            pltpu.sync_copy(sidx_hbm.at[crow0 + c], sring.at[c])
        plsc.subcore_barrier()

        def G(c, rbuf, gsem, abuf, asem):
            pltpu.async_copy(table_hbm.at[gring.at[c % 4]], rbuf, gsem)
            if weighted:
                pltpu.async_copy(ad_hbm.at[sring.at[c % 4]], abuf, asem)

        def WG(rbuf, gsem, abuf, asem):
            pltpu.make_async_copy(
                table_hbm.at[gring.at[0]], rbuf, gsem).wait()
            if weighted:
                pltpu.make_async_copy(
                    ad_hbm.at[sring.at[0]], abuf, asem).wait()

        def S(c, rbuf, sem):
            pltpu.async_copy(rbuf, acc_sh.at[sring.at[c % 4]], sem, add=True)

        def WS(rbuf, sem):
            pltpu.make_async_copy(rbuf, acc_sh.at[sring.at[0]], sem).wait()

        def I(c, isem):
            pltpu.async_copy(gidx_hbm.at[crow0 + c], gring.at[c % 4], isem)
            pltpu.async_copy(sidx_hbm.at[crow0 + c], sring.at[c % 4], isem)

        def WI(isem):
            pltpu.make_async_copy(gidx_hbm.at[crow0], gring.at[0], isem).wait()
            pltpu.make_async_copy(sidx_hbm.at[crow0], sring.at[0], isem).wait()

        def compute(rbuf, abuf):
            if not weighted:
                return
            for g in range(k // 16):
                rid = g * 16 + jnp.arange(16, dtype=jnp.int32)
                a_s = plsc.load_gather(
                    rbuf, [rid, jnp.full((16,), C1, jnp.int32)])
                a_d = plsc.load_gather(
                    abuf, [rid, jnp.zeros((16,), jnp.int32)])
                e = a_s + a_d
                e = jnp.where(e > 0, e, 0.01 * e)
                ex16 = jnp.exp(e)
                for l in range(16):
                    r = g * 16 + l
                    svec = jnp.full((16,), ex16[l])
                    for j in range(width // 16):
                        rbuf[r, pl.ds(j * 16, 16)] = (
                            rbuf[r, pl.ds(j * 16, 16)] * svec)

        G(0, rows0, gsem0, adr0, asem0)

        def pair(i, carry):
            c0 = i * 2
            WG(rows0, gsem0, adr0, asem0)

            @pl.when(i > 0)
            def _():
                WS(rows1, ssem1)

            @pl.when(c0 + 3 < ncs)
            def _():
                I(c0 + 3, isem1)

            @pl.when(i > 0)
            def _():
                WI(isem1)

            G(c0 + 1, rows1, gsem1, adr1, asem1)
            compute(rows0, adr0)
            S(c0, rows0, ssem0)
            WG(rows1, gsem1, adr1, asem1)
            WS(rows0, ssem0)

            @pl.when(c0 + 4 < ncs)
            def _():
                I(c0 + 4, isem0)

            @pl.when((i > 0) & (c0 + 2 < ncs))
            def _():
                WI(isem0)

            @pl.when(c0 + 2 < ncs)
            def _():
                G(c0 + 2, rows0, gsem0, adr0, asem0)

            compute(rows1, adr1)
            S(c0 + 1, rows1, ssem1)
            return carry

        lax.fori_loop(0, npair, pair, 0)
        if tail:
            WG(rows0, gsem0, adr0, asem0)
            WS(rows1, ssem1)
            compute(rows0, adr0)
            S(ncs - 1, rows0, ssem0)
            WS(rows0, ssem0)
        else:
            WS(rows1, ssem1)
        plsc.subcore_barrier()
        pltpu.sync_copy(acc_sh.at[pl.ds(sid * RPT, RPT)],
                        out_hbm.at[cid, pl.ds(sid * RPT, RPT)])

    kern = functools.partial(
        pl.kernel,
        mesh=mesh,
        out_type=jax.ShapeDtypeStruct((NC, NPAD, width), jnp.float32),
        compiler_params=_SC_PARAMS,
        scratch_types=scratch,
    )(body)
    zeros = jnp.zeros((RPT, width), jnp.float32)
    if weighted:
        return kern(table, ad, gidx2, sidx2, zeros)
    return kern(table, gidx2, sidx2, zeros)


def _tc1_body(feat, w, b, out):
    h = jnp.dot(feat[...], w[...], preferred_element_type=jnp.float32) + b[...]
    ones = jnp.ones((_BLK, 1), jnp.float32)
    zpad = jnp.zeros((_BLK, WH - D - 1), jnp.float32)
    out[...] = jnp.concatenate([h, ones, zpad], axis=1)


def _tc2_body(acc, h1a, we, be, x_out, h2_out, deg_out):
    agg = acc[0, :, :D] + acc[1, :, :D]
    deg = acc[0, :, D:D + 1] + acc[1, :, D:D + 1]
    h1 = h1a[:, :D]
    xm = jnp.where(deg > 0, agg / jnp.maximum(deg, 1.0), h1)
    nrm = jnp.sqrt(jnp.sum(xm * xm, axis=1, keepdims=True))
    x = xm / jnp.maximum(nrm, 1e-12)
    x_out[...] = x
    h2_out[...] = jnp.dot(x, we[...], preferred_element_type=jnp.float32) + be[...]
    deg_out[...] = deg


def _tc3_body(acc, deg_in, h2, wa, ba, wsrc, wdst, batt, z_out, haug_out, ad_out):
    deg = deg_in[...]
    agg = acc[0] + acc[1]
    z = jnp.where(deg > 0, agg / jnp.maximum(deg, 1.0), h2[...])
    z_out[...] = z
    ha = jnp.dot(z, wa[...], preferred_element_type=jnp.float32) + ba[...]
    a_s = jnp.dot(ha, wsrc[...], preferred_element_type=jnp.float32)
    a_d = jnp.dot(ha, wdst[...], preferred_element_type=jnp.float32) + batt[...]
    ones = jnp.ones((_BLK, 1), jnp.float32)
    zpad = jnp.zeros((_BLK, WA - C1 - 2), jnp.float32)
    haug_out[...] = jnp.concatenate([ha, a_s, ones, zpad], axis=1)
    ad_out[...] = jnp.concatenate([a_d, jnp.zeros((_BLK, 15), jnp.float32)], axis=1)


def _tc4_body(acc, deg_in, haug, spad_out):
    deg = deg_in[...]
    num = acc[0, :, :C1] + acc[1, :, :C1]
    den = acc[0, :, C1 + 1:C1 + 2] + acc[1, :, C1 + 1:C1 + 2]
    ha = haug[:, :C1]
    att = jnp.where(deg > 0, num / jnp.where(den > 0, den, 1.0), ha)
    m = jnp.max(att, axis=1, keepdims=True)
    ex = jnp.exp(att - m)
    s = ex / jnp.sum(ex, axis=1, keepdims=True)
    spad_out[...] = jnp.concatenate(
        [s, jnp.zeros((_BLK, WA - C1), jnp.float32)], axis=1)


def _tc4b_body(spad, z, tacc, x2_out, adj_out):
    i = pl.program_id(0)

    @pl.when(i == 0)
    def _():
        x2_out[...] = jnp.zeros_like(x2_out)
        adj_out[...] = jnp.zeros_like(adj_out)

    s_blk = spad[...]
    t_blk = tacc[0] + tacc[1]
    dn = (((0,), (0,)), ((), ()))
    x2_out[...] += lax.dot_general(s_blk, z[...], dn,
                                   preferred_element_type=jnp.float32)
    adj_out[...] += lax.dot_general(s_blk, t_blk, dn,
                                    preferred_element_type=jnp.float32)


def _tc5_body(x2, adj_in, we2, be2, wa2, ba2, w2s, w2d, batt2,
              s2_out, x3_out, emb_out):
    adj = adj_in[...]
    ii = lax.broadcasted_iota(jnp.int32, (C1, C1), 0)
    jj = lax.broadcasted_iota(jnp.int32, (C1, C1), 1)
    eye = (ii == jj)
    adj = jnp.where(eye, 0.0, adj)
    m1 = jnp.maximum((adj != 0).astype(jnp.float32),
                     eye.astype(jnp.float32))
    h = jnp.dot(x2[...], we2[...], preferred_element_type=jnp.float32) + be2[...]
    indeg = jnp.sum(m1, axis=0, keepdims=True)
    dn = (((0,), (0,)), ((), ()))
    aggd = lax.dot_general(m1, h, dn, preferred_element_type=jnp.float32)
    indeg_c = indeg.reshape(C1, 1)
    z2 = jnp.where(indeg_c > 0, aggd / jnp.maximum(indeg_c, 1.0), h)
    ha2 = jnp.dot(z2, wa2[...], preferred_element_type=jnp.float32) + ba2[...]
    a_s = jnp.dot(ha2, w2s[...], preferred_element_type=jnp.float32)
    a_d = jnp.dot(ha2, w2d[...], preferred_element_type=jnp.float32)
    a = a_s + a_d.reshape(1, C1) + batt2[...]
    e = jnp.where(a > 0, a, 0.01 * a)
    e = jnp.where(m1 > 0, e, -1e9)
    mcol = jnp.max(e, axis=0, keepdims=True)
    exv = jnp.exp(e - mcol)
    alpha = exv / jnp.sum(exv, axis=0, keepdims=True)
    outa = lax.dot_general(alpha, ha2, dn, preferred_element_type=jnp.float32)
    att2 = jnp.where(indeg_c > 0, outa, ha2)
    m2 = jnp.max(att2, axis=1, keepdims=True)
    ex2 = jnp.exp(att2 - m2)
    s2 = ex2 / jnp.sum(ex2, axis=1, keepdims=True)
    s2_out[...] = s2
    x3 = lax.dot_general(s2, z2, dn, preferred_element_type=jnp.float32)
    x3_out[...] = x3
    emb_out[...] = jnp.mean(x3).reshape(1, 1)


def _row_spec(width):
    return pl.BlockSpec((_BLK, width), lambda i: (i, 0))


def _full_spec(shape):
    nd = len(shape)
    return pl.BlockSpec(shape, lambda i, _n=nd: (0,) * _n)


def _acc_spec(width):
    return pl.BlockSpec((NC, _BLK, width), lambda i: (0, i, 0))


def kernel(feature, edge_index, W_f1, b_f1, W_e1, b_e1, W_a1, b_a1, watt1,
           batt1, W_e2, b_e2, W_a2, b_a2, watt2, batt2):
    src = edge_index[0]
    dst = edge_index[1]
    src2s = src.reshape(E // KS, KS)
    dst2s = dst.reshape(E // KS, KS)
    src2a = src.reshape(E // KA, KA)
    dst2a = dst.reshape(E // KA, KA)

    h1a = pl.pallas_call(
        _tc1_body,
        grid=(_NBLK,),
        in_specs=[_row_spec(D), _full_spec((D, D)), _full_spec((1, D))],
        out_specs=_row_spec(WH),
        out_shape=jax.ShapeDtypeStruct((N, WH), jnp.float32),
    )(feature, W_f1, b_f1.reshape(1, D))

    accA = _edge_pass(h1a, src2s, dst2s, WH, KS)

    x, h2, deg = pl.pallas_call(
        _tc2_body,
        grid=(_NBLK,),
        in_specs=[_acc_spec(WH), _row_spec(WH), _full_spec((D, D)),
                  _full_spec((1, D))],
        out_specs=[_row_spec(D), _row_spec(D), _row_spec(1)],
        out_shape=[jax.ShapeDtypeStruct((N, D), jnp.float32),
                   jax.ShapeDtypeStruct((N, D), jnp.float32),
                   jax.ShapeDtypeStruct((N, 1), jnp.float32)],
    )(accA, h1a, W_e1, b_e1.reshape(1, D))

    accB = _edge_pass(h2, src2s, dst2s, D, KS)

    z, ha_aug, ad_pad = pl.pallas_call(
        _tc3_body,
        grid=(_NBLK,),
        in_specs=[_acc_spec(D), _row_spec(1), _row_spec(D),
                  _full_spec((D, C1)), _full_spec((1, C1)),
                  _full_spec((C1, 1)), _full_spec((C1, 1)),
                  _full_spec((1, 1))],
        out_specs=[_row_spec(D), _row_spec(WA), _row_spec(16)],
        out_shape=[jax.ShapeDtypeStruct((N, D), jnp.float32),
                   jax.ShapeDtypeStruct((N, WA), jnp.float32),
                   jax.ShapeDtypeStruct((N, 16), jnp.float32)],
    )(accB, deg, h2, W_a1, b_a1.reshape(1, C1),
      watt1[:C1].reshape(C1, 1), watt1[C1:].reshape(C1, 1),
      batt1.reshape(1, 1))

    accC = _edge_pass(ha_aug, src2a, dst2a, WA, KA, ad=ad_pad)

    s_pad = pl.pallas_call(
        _tc4_body,
        grid=(_NBLK,),
        in_specs=[_acc_spec(WA), _row_spec(1), _row_spec(WA)],
        out_specs=_row_spec(WA),
        out_shape=jax.ShapeDtypeStruct((N, WA), jnp.float32),
    )(accC, deg, ha_aug)

    accT = _edge_pass(s_pad, dst2s, src2s, WA, KS)

    x2f, adjf = pl.pallas_call(
        _tc4b_body,
        grid=(_NBLK,),
        in_specs=[_row_spec(WA), _row_spec(D), _acc_spec(WA)],
        out_specs=[_full_spec((WA, D)), _full_spec((WA, WA))],
        out_shape=[jax.ShapeDtypeStruct((WA, D), jnp.float32),
                   jax.ShapeDtypeStruct((WA, WA), jnp.float32)],
    )(s_pad, z, accT)

    s2, x3, emb = pl.pallas_call(
        _tc5_body,
        grid=(1,),
        in_specs=[_full_spec((C1, D)), _full_spec((C1, C1)),
                  _full_spec((D, D)), _full_spec((1, D)),
                  _full_spec((D, C2)), _full_spec((1, C2)),
                  _full_spec((C2, 1)), _full_spec((C2, 1)),
                  _full_spec((1, 1))],
        out_specs=[_full_spec((C1, C2)), _full_spec((C2, D)),
                   _full_spec((1, 1))],
        out_shape=[jax.ShapeDtypeStruct((C1, C2), jnp.float32),
                   jax.ShapeDtypeStruct((C2, D), jnp.float32),
                   jax.ShapeDtypeStruct((1, 1), jnp.float32)],
    )(x2f[:C1], adjf[:C1, :C1], W_e2, b_e2.reshape(1, D),
      W_a2, b_a2.reshape(1, C2), watt2[:C2].reshape(C2, 1),
      watt2[C2:].reshape(C2, 1), batt2.reshape(1, 1))

    s = s_pad[:, :C1]
    x2 = x2f[:C1]
    assign1 = jnp.ones((C2, 1), jnp.float32)
    return (s, s2, assign1, x, x2, x3, emb[0, 0])
